# baseline (speedup 1.0000x reference)
import jax
import jax.numpy as jnp
import numpy as np
from jax import lax
from jax.experimental import pallas as pl
from jax.experimental.pallas import tpu as pltpu

_KS0 = np.int32(0)
_KS1 = np.int32(42)
_KS2 = np.int32(np.uint32(0x1BD11BDA ^ 42).view(np.int32))
_ROTS_A = (13, 15, 26, 6)
_ROTS_B = (17, 29, 16, 24)
_THRESH = np.int32(419431)


def _keep_bits(j):
    x0 = jnp.zeros_like(j)
    x1 = j + _KS1
    inj = ((_KS1, _KS2, 1), (_KS2, _KS0, 2), (_KS0, _KS1, 3),
           (_KS1, _KS2, 4), (_KS2, _KS0, 5))
    for i, (ka, kb, cnt) in enumerate(inj):
        for r in (_ROTS_A if i % 2 == 0 else _ROTS_B):
            x0 = x0 + x1
            x1 = (x1 << r) | lax.shift_right_logical(x1, 32 - r)
            x1 = x1 ^ x0
        x0 = x0 + ka
        x1 = x1 + jnp.int32(kb + np.int32(cnt))
    return lax.shift_right_logical(x0 ^ x1, 9) >= _THRESH


def _body(x_in, e_in, y_in, tr_in, te_in,
          x_out, e_out, y_out, tr_out, te_out,
          sem_x, sem_e, sem_y):
    cp_x = pltpu.make_async_copy(x_in, x_out, sem_x)
    cp_e = pltpu.make_async_copy(e_in, e_out, sem_e)
    cp_y = pltpu.make_async_copy(y_in, y_out, sem_y)
    cp_x.start()
    cp_e.start()
    cp_y.start()

    n = tr_in.shape[0]
    keep = _keep_bits(lax.broadcasted_iota(jnp.int32, (n,), 0))
    tr_out[...] = keep & tr_in[...]
    te_out[...] = keep & te_in[...]

    cp_x.wait()
    cp_e.wait()
    cp_y.wait()


def _make(n, d, e):
    any_spec = pl.BlockSpec(memory_space=pl.ANY)
    vmem_spec = pl.BlockSpec(memory_space=pltpu.MemorySpace.VMEM)
    return pl.pallas_call(
        _body,
        in_specs=[any_spec, any_spec, any_spec, vmem_spec, vmem_spec],
        out_specs=[any_spec, any_spec, any_spec, vmem_spec, vmem_spec],
        out_shape=[
            jax.ShapeDtypeStruct((n, d), jnp.float32),
            jax.ShapeDtypeStruct((2, e), jnp.int32),
            jax.ShapeDtypeStruct((n,), jnp.int32),
            jax.ShapeDtypeStruct((n,), jnp.bool_),
            jax.ShapeDtypeStruct((n,), jnp.bool_),
        ],
        scratch_shapes=[pltpu.SemaphoreType.DMA] * 3,
    )


def kernel(x, y, train_mask, test_mask, edge_index):
    n, d = x.shape
    e = edge_index.shape[1]
    x_o, e_o, y_o, tr_o, te_o = _make(n, d, e)(
        x, edge_index, y, train_mask, test_mask)
    return (x_o, e_o, y_o, tr_o, te_o)
